# baseline (speedup 1.0000x reference)
import jax
import jax.numpy as jnp
from jax import lax
from jax.experimental import pallas as pl
from jax.experimental.pallas import tpu as pltpu

_B = 16
_SUB = 8
_ROW = 3584
_F = 7
_UNIFORM = 0.1429
_EPS = 1e-5


def _shl(x, k):
    pad = jnp.zeros((x.shape[0], k), dtype=x.dtype)
    return jnp.concatenate([x[:, k:], pad], axis=1)


def _shr(x, k):
    pad = jnp.zeros((x.shape[0], k), dtype=x.dtype)
    return jnp.concatenate([pad, x[:, : x.shape[1] - k]], axis=1)


def _body(len_ref, t_ref, y_ref, out_ref, acc_ref):
    b = pl.program_id(0)

    @pl.when(b == 0)
    def _init():
        acc_ref[0] = 0.0
        acc_ref[1] = 0.0

    t = t_ref[0]
    y = y_ref[0]

    rd = lax.round(t * 10000.0, lax.RoundingMethod.TO_NEAREST_EVEN) / 10000.0
    eqf = jnp.where(rd == jnp.float32(_UNIFORM), 1.0, 0.0).astype(jnp.float32)

    s1 = eqf + _shl(eqf, 1)
    s2 = s1 + _shl(s1, 2)
    g = s2 + _shl(s1, 4) + _shl(eqf, 6)

    lane = lax.broadcasted_iota(jnp.int32, (_SUB, _ROW), 1)
    row = lax.broadcasted_iota(jnp.int32, (_SUB, _ROW), 0)
    flat = row * _ROW + lane
    lenb = len_ref[b]
    start = (lane % _F) == 0
    valid = flat < _F * lenb
    cap = jnp.where(start & (g != 7.0) & valid, 1.0, 0.0).astype(jnp.float32)

    e1 = cap + _shr(cap, 1)
    e2 = e1 + _shr(e1, 2)
    m = e2 + _shr(e1, 4) + _shr(cap, 6)

    xlogy = jnp.where(t > 0.0, t * jnp.log(t), 0.0)
    per = xlogy - t * y
    acc_ref[0] += jnp.sum(per * m)
    acc_ref[1] += jnp.where(jnp.sum(cap) > 0.0, 1.0, 0.0)

    @pl.when(b == pl.num_programs(0) - 1)
    def _fin():
        out_ref[0, 0] = (jnp.float32(_EPS) + acc_ref[0]) / acc_ref[1]


def kernel(target, output, length):
    B, T, F = target.shape
    t3 = target.reshape(B, _SUB, _ROW)
    y3 = output.reshape(B, _SUB, _ROW)
    out = pl.pallas_call(
        _body,
        grid=(B,),
        in_specs=[
            pl.BlockSpec(memory_space=pltpu.SMEM),
            pl.BlockSpec((1, _SUB, _ROW), lambda b: (b, 0, 0)),
            pl.BlockSpec((1, _SUB, _ROW), lambda b: (b, 0, 0)),
        ],
        out_specs=pl.BlockSpec(memory_space=pltpu.SMEM),
        out_shape=jax.ShapeDtypeStruct((1, 1), jnp.float32),
        scratch_shapes=[pltpu.SMEM((2,), jnp.float32)],
    )(length.astype(jnp.int32), t3, y3)
    return out[0, 0]
